# baseline (speedup 1.0000x reference)
import functools

import jax
import jax.numpy as jnp
from jax import lax
from jax.experimental import pallas as pl
from jax.experimental.pallas import tpu as pltpu
from jax.experimental.pallas import tpu_sc as plsc

_BLK = 128
_WPB = 2 * _BLK


def _proj_body(h_ref, w_ref, b_ref, o_ref):
    o_ref[...] = (
        lax.dot_general(
            w_ref[...],
            h_ref[...],
            (((0,), (1,)), ((), ())),
            preferred_element_type=jnp.float32,
        )
        + b_ref[...]
    )


def _make_edge_kernel(n_nodes, n_edges, nc, ns, lanes):
    nw = nc * ns
    nblk = n_edges // _BLK
    nb = nblk // nw
    extra = nblk - nb * nw
    mesh = plsc.VectorSubcoreMesh(core_axis_name="c", subcore_axis_name="s")

    @functools.partial(
        pl.kernel,
        mesh=mesh,
        out_type=jax.ShapeDtypeStruct((2 * n_edges,), jnp.float32),
        scratch_types=[
            pltpu.VMEM((4 * n_nodes,), jnp.float32),
            pltpu.VMEM((nb * _BLK,), jnp.int32),
            pltpu.VMEM((nb * _BLK,), jnp.int32),
            pltpu.VMEM((nb * _WPB,), jnp.float32),
            pltpu.VMEM((_BLK,), jnp.int32),
            pltpu.VMEM((_BLK,), jnp.int32),
            pltpu.VMEM((_WPB,), jnp.float32),
        ],
        compiler_params=pltpu.CompilerParams(needs_layout_passes=False),
    )
    def edge_kernel(p_hbm, ei_hbm, out_hbm, tbl, ivs, ivd, outv, ivse, ivde, outve):
        wid = lax.axis_index("s") * nc + lax.axis_index("c")
        base_e = wid * (nb * _BLK)
        base_w = wid * (nb * _WPB)
        pltpu.sync_copy(p_hbm, tbl)
        pltpu.sync_copy(ei_hbm.at[0, pl.ds(base_e, nb * _BLK)], ivs)
        pltpu.sync_copy(ei_hbm.at[1, pl.ds(base_e, nb * _BLK)], ivd)

        def step(ivs_ref, ivd_ref, outv_ref, soff, loff):
            src = ivs_ref[pl.ds(soff // 2 + loff, lanes)]
            dst = ivd_ref[pl.ds(soff // 2 + loff, lanes)]
            a0 = plsc.load_gather(tbl, [src])
            a1 = plsc.load_gather(tbl, [src + n_nodes])
            c0 = plsc.load_gather(tbl, [dst + 2 * n_nodes])
            c1 = plsc.load_gather(tbl, [dst + 3 * n_nodes])
            y0 = 1.0 / (1.0 + jnp.exp(-(a0 + c0)))
            y1 = 1.0 / (1.0 + jnp.exp(-(a1 + c1)))
            outv_ref[pl.ds(soff + loff, lanes)] = y0
            outv_ref[pl.ds(soff + _BLK + loff, lanes)] = y1

        def block_body(k, carry):
            soff = k * (2 * _WPB)
            for u in range(2):
                for loff in range(0, _BLK, lanes):
                    step(ivs, ivd, outv, soff + u * _WPB, loff)
            return carry

        lax.fori_loop(0, nb // 2, block_body, 0)
        pltpu.sync_copy(outv, out_hbm.at[pl.ds(base_w, nb * _WPB)])

        @pl.when(wid < extra)
        def _tail():
            tail_e = (nblk - extra + wid) * _BLK
            tail_w = (nblk - extra + wid) * _WPB
            pltpu.sync_copy(ei_hbm.at[0, pl.ds(tail_e, _BLK)], ivse)
            pltpu.sync_copy(ei_hbm.at[1, pl.ds(tail_e, _BLK)], ivde)
            for loff in range(0, _BLK, lanes):
                step(ivse, ivde, outve, 0, loff)
            pltpu.sync_copy(outve, out_hbm.at[pl.ds(tail_w, _WPB)])

    return edge_kernel


def kernel(h, edge_index, W, b):
    n_nodes, d = h.shape
    n_edges = edge_index.shape[1]
    n_classes = b.shape[0]
    nblk = n_edges // _BLK

    wcat = jnp.concatenate([W[:d], W[d:]], axis=1)
    bcat = jnp.concatenate([b, jnp.zeros_like(b)])[:, None]

    p = pl.pallas_call(
        _proj_body,
        out_shape=jax.ShapeDtypeStruct((2 * n_classes, n_nodes), jnp.float32),
    )(h, wcat, bcat)

    info = plsc.get_sparse_core_info()
    edge_fn = _make_edge_kernel(
        n_nodes, n_edges, info.num_cores, info.num_subcores, info.num_lanes
    )
    ei = edge_index.astype(jnp.int32)
    out_flat = edge_fn(p.reshape(-1), ei)
    return (
        out_flat.reshape(nblk, n_classes, _BLK)
        .transpose(0, 2, 1)
        .reshape(n_edges, n_classes)
    )


# speedup vs baseline: 1.4132x; 1.4132x over previous
import functools

import jax
import jax.numpy as jnp
from jax import lax
from jax.experimental import pallas as pl
from jax.experimental.pallas import tpu as pltpu
from jax.experimental.pallas import tpu_sc as plsc

_BLK = 128
_WPB = 2 * _BLK


def _proj_body(h_ref, w_ref, b_ref, o_ref):
    o_ref[...] = (
        lax.dot_general(
            w_ref[...],
            h_ref[...],
            (((0,), (1,)), ((), ())),
            preferred_element_type=jnp.float32,
        )
        + b_ref[...]
    )


def _make_edge_kernel(n_nodes, n_edges, nc, ns, lanes):
    nw = nc * ns
    nblk = n_edges // _BLK
    nb = nblk // nw
    extra = nblk - nb * nw
    mesh = plsc.VectorSubcoreMesh(core_axis_name="c", subcore_axis_name="s")

    @functools.partial(
        pl.kernel,
        mesh=mesh,
        out_type=jax.ShapeDtypeStruct((2 * n_edges,), jnp.float32),
        scratch_types=[
            pltpu.VMEM((4 * n_nodes,), jnp.float32),
            pltpu.VMEM((nb * _BLK,), jnp.int32),
            pltpu.VMEM((nb * _BLK,), jnp.int32),
            pltpu.VMEM((nb * _WPB,), jnp.float32),
            pltpu.VMEM((_BLK,), jnp.int32),
            pltpu.VMEM((_BLK,), jnp.int32),
            pltpu.VMEM((_WPB,), jnp.float32),
        ],
        compiler_params=pltpu.CompilerParams(needs_layout_passes=False),
    )
    def edge_kernel(p_hbm, ei_hbm, out_hbm, tbl, ivs, ivd, outv, ivse, ivde, outve):
        wid = lax.axis_index("s") * nc + lax.axis_index("c")
        base_e = wid * (nb * _BLK)
        base_w = wid * (nb * _WPB)
        pltpu.sync_copy(p_hbm, tbl)
        pltpu.sync_copy(ei_hbm.at[0, pl.ds(base_e, nb * _BLK)], ivs)
        pltpu.sync_copy(ei_hbm.at[1, pl.ds(base_e, nb * _BLK)], ivd)

        def steps(ivs_ref, ivd_ref, outv_ref, soff, loffs):
            ioff = soff // 2
            ss = [ivs_ref[pl.ds(ioff + l, lanes)] for l in loffs]
            ds = [ivd_ref[pl.ds(ioff + l, lanes)] for l in loffs]
            a0s = [plsc.load_gather(tbl, [s]) for s in ss]
            c0s = [plsc.load_gather(tbl, [d + 2 * n_nodes]) for d in ds]
            a1s = [plsc.load_gather(tbl, [s + n_nodes]) for s in ss]
            c1s = [plsc.load_gather(tbl, [d + 3 * n_nodes]) for d in ds]
            x0s = [a + c for a, c in zip(a0s, c0s)]
            x1s = [a + c for a, c in zip(a1s, c1s)]
            e0s = [jnp.exp(x) for x in x0s]
            e1s = [jnp.exp(x) for x in x1s]
            y0s = [1.0 / (1.0 + e) for e in e0s]
            y1s = [1.0 / (1.0 + e) for e in e1s]
            for l, y in zip(loffs, y0s):
                outv_ref[pl.ds(soff + l, lanes)] = y
            for l, y in zip(loffs, y1s):
                outv_ref[pl.ds(soff + _BLK + l, lanes)] = y

        def block_body(k, carry):
            soff = k * _WPB
            steps(ivs, ivd, outv, soff, [l * lanes for l in range(4)])
            steps(ivs, ivd, outv, soff, [l * lanes for l in range(4, 8)])
            return carry

        lax.fori_loop(0, nb, block_body, 0)
        pltpu.sync_copy(outv, out_hbm.at[pl.ds(base_w, nb * _WPB)])

        @pl.when(wid < extra)
        def _tail():
            tail_e = (nblk - extra + wid) * _BLK
            tail_w = (nblk - extra + wid) * _WPB
            pltpu.sync_copy(ei_hbm.at[0, pl.ds(tail_e, _BLK)], ivse)
            pltpu.sync_copy(ei_hbm.at[1, pl.ds(tail_e, _BLK)], ivde)
            steps(ivse, ivde, outve, 0, [l * lanes for l in range(4)])
            steps(ivse, ivde, outve, 0, [l * lanes for l in range(4, 8)])
            pltpu.sync_copy(outve, out_hbm.at[pl.ds(tail_w, _WPB)])

    return edge_kernel


def kernel(h, edge_index, W, b):
    n_nodes, d = h.shape
    n_edges = edge_index.shape[1]
    n_classes = b.shape[0]
    nblk = n_edges // _BLK

    wcat = -jnp.concatenate([W[:d], W[d:]], axis=1)
    bcat = -jnp.concatenate([b, jnp.zeros_like(b)])[:, None]

    p = pl.pallas_call(
        _proj_body,
        out_shape=jax.ShapeDtypeStruct((2 * n_classes, n_nodes), jnp.float32),
    )(h, wcat, bcat)

    info = plsc.get_sparse_core_info()
    edge_fn = _make_edge_kernel(
        n_nodes, n_edges, info.num_cores, info.num_subcores, info.num_lanes
    )
    ei = edge_index.astype(jnp.int32)
    out_flat = edge_fn(p.reshape(-1), ei)
    return (
        out_flat.reshape(nblk, n_classes, _BLK)
        .transpose(0, 2, 1)
        .reshape(n_edges, n_classes)
    )


# speedup vs baseline: 1.4665x; 1.0377x over previous
import functools

import jax
import jax.numpy as jnp
from jax import lax
from jax.experimental import pallas as pl
from jax.experimental.pallas import tpu as pltpu
from jax.experimental.pallas import tpu_sc as plsc

_BLK = 128
_WPB = 2 * _BLK


def _proj_body(h_ref, w_ref, b_ref, o_ref):
    o_ref[...] = (
        lax.dot_general(
            w_ref[...],
            h_ref[...],
            (((0,), (1,)), ((), ())),
            preferred_element_type=jnp.float32,
        )
        + b_ref[...]
    )


def _make_edge_kernel(n_nodes, n_edges, nc, ns, lanes):
    nw = nc * ns
    nblk = n_edges // _BLK
    nb = nblk // nw
    extra = nblk - nb * nw
    mesh = plsc.VectorSubcoreMesh(core_axis_name="c", subcore_axis_name="s")

    @functools.partial(
        pl.kernel,
        mesh=mesh,
        out_type=jax.ShapeDtypeStruct((2 * n_edges,), jnp.float32),
        scratch_types=[
            pltpu.VMEM((4 * n_nodes,), jnp.float32),
            pltpu.VMEM((nb * _BLK,), jnp.int32),
            pltpu.VMEM((nb * _BLK,), jnp.int32),
            pltpu.VMEM((nb * _WPB,), jnp.float32),
            pltpu.VMEM((_BLK,), jnp.int32),
            pltpu.VMEM((_BLK,), jnp.int32),
            pltpu.VMEM((_WPB,), jnp.float32),
        ],
        compiler_params=pltpu.CompilerParams(needs_layout_passes=False),
    )
    def edge_kernel(p_hbm, ei_hbm, out_hbm, tbl, ivs, ivd, outv, ivse, ivde, outve):
        wid = lax.axis_index("s") * nc + lax.axis_index("c")
        base_e = wid * (nb * _BLK)
        base_w = wid * (nb * _WPB)
        pltpu.sync_copy(p_hbm, tbl)
        pltpu.sync_copy(ei_hbm.at[0, pl.ds(base_e, nb * _BLK)], ivs)
        pltpu.sync_copy(ei_hbm.at[1, pl.ds(base_e, nb * _BLK)], ivd)

        def steps(ivs_ref, ivd_ref, outv_ref, soff, loffs):
            ioff = soff // 2
            ss = [ivs_ref[pl.ds(ioff + l, lanes)] for l in loffs]
            ds = [ivd_ref[pl.ds(ioff + l, lanes)] for l in loffs]
            a0s = [plsc.load_gather(tbl, [s]) for s in ss]
            c0s = [plsc.load_gather(tbl, [d + 2 * n_nodes]) for d in ds]
            a1s = [plsc.load_gather(tbl, [s + n_nodes]) for s in ss]
            c1s = [plsc.load_gather(tbl, [d + 3 * n_nodes]) for d in ds]
            x0s = [a + c for a, c in zip(a0s, c0s)]
            x1s = [a + c for a, c in zip(a1s, c1s)]
            e0s = [jnp.exp(x) for x in x0s]
            e1s = [jnp.exp(x) for x in x1s]
            y0s = [1.0 / (1.0 + e) for e in e0s]
            y1s = [1.0 / (1.0 + e) for e in e1s]
            for l, y in zip(loffs, y0s):
                outv_ref[pl.ds(soff + l, lanes)] = y
            for l, y in zip(loffs, y1s):
                outv_ref[pl.ds(soff + _BLK + l, lanes)] = y

        def block_body(k, carry):
            soff = k * _WPB
            steps(ivs, ivd, outv, soff, [l * lanes for l in range(8)])
            return carry

        lax.fori_loop(0, nb, block_body, 0)
        pltpu.sync_copy(outv, out_hbm.at[pl.ds(base_w, nb * _WPB)])

        @pl.when(wid < extra)
        def _tail():
            tail_e = (nblk - extra + wid) * _BLK
            tail_w = (nblk - extra + wid) * _WPB
            pltpu.sync_copy(ei_hbm.at[0, pl.ds(tail_e, _BLK)], ivse)
            pltpu.sync_copy(ei_hbm.at[1, pl.ds(tail_e, _BLK)], ivde)
            steps(ivse, ivde, outve, 0, [l * lanes for l in range(8)])
            pltpu.sync_copy(outve, out_hbm.at[pl.ds(tail_w, _WPB)])

    return edge_kernel


def kernel(h, edge_index, W, b):
    n_nodes, d = h.shape
    n_edges = edge_index.shape[1]
    n_classes = b.shape[0]
    nblk = n_edges // _BLK

    wcat = -jnp.concatenate([W[:d], W[d:]], axis=1)
    bcat = -jnp.concatenate([b, jnp.zeros_like(b)])[:, None]

    p = pl.pallas_call(
        _proj_body,
        out_shape=jax.ShapeDtypeStruct((2 * n_classes, n_nodes), jnp.float32),
    )(h, wcat, bcat)

    info = plsc.get_sparse_core_info()
    edge_fn = _make_edge_kernel(
        n_nodes, n_edges, info.num_cores, info.num_subcores, info.num_lanes
    )
    ei = edge_index.astype(jnp.int32)
    out_flat = edge_fn(p.reshape(-1), ei)
    return (
        out_flat.reshape(nblk, n_classes, _BLK)
        .transpose(0, 2, 1)
        .reshape(n_edges, n_classes)
    )
